# baseline (speedup 1.0000x reference)
import functools

import jax
import jax.numpy as jnp
from jax import lax
from jax.experimental import pallas as pl
from jax.experimental.pallas import tpu as pltpu
from jax.experimental.pallas import tpu_sc as plsc

B = 8
HIS = 13
NN = 1250
S = 5
IN_DIM = 128
HID = 64
PRED = 12
E = 800000
NTOT = B * S * NN
HALF = HID // 2
TOTEL = NTOT * HID

NSC = 16
CHUNK = 128
KP = 4
GSZ = KP * CHUNK
G = 98
EPT = G * GSZ
EP = NSC * EPT
PAD = EP - E
EXT = 2 * GSZ
KD = 8
GD = EPT // (KD * CHUNK)
ACCR = NTOT + 8
ROWCH = NTOT // CHUNK
ROW_TAIL = NTOT - ROWCH * CHUNK
ZTAIL = ACCR - ROWCH * CHUNK
ROW_ITERS = -(-ROWCH // NSC)
ZCH = 2000
NZCH = NTOT // ZCH

NF = NTOT // 2
XF = B * HIS * NN // 2
XBLK = 5000
BLK = 5000
NBLK = NF // BLK

_mesh = plsc.VectorSubcoreMesh(core_axis_name="c", subcore_axis_name="s")


@functools.partial(
    pl.kernel,
    out_type=jax.ShapeDtypeStruct((2 * NTOT,), jnp.float32),
    mesh=_mesh,
    compiler_params=pltpu.CompilerParams(use_tc_tiling_on_sc=False),
    scratch_types=[
        pltpu.VMEM((KD, CHUNK), jnp.int32),
        pltpu.VMEM((1, CHUNK), jnp.float32),
        pltpu.VMEM((ZCH,), jnp.float32),
        pltpu.VMEM_SHARED((ACCR,), jnp.float32),
        pltpu.SemaphoreType.DMA,
        pltpu.SemaphoreType.DMA,
    ],
)
def _sc_degrees(edges_hbm, zeros_hbm, out_hbm, idx, ones_v, vbuf, acc,
                semi, sema):
    c = lax.axis_index("c")
    s = lax.axis_index("s")
    for i in range(CHUNK // 16):
        ones_v[0, pl.ds(i * 16, 16)] = jnp.full((16,), 1.0, jnp.float32)
    pltpu.sync_copy(zeros_hbm, vbuf)

    def zbody(j, carry):
        cid = j * NSC + s

        @pl.when(cid < NZCH)
        def _():
            pltpu.sync_copy(vbuf, acc.at[pl.ds(cid * ZCH, ZCH)])

        return carry

    lax.fori_loop(0, -(-NZCH // NSC), zbody, None)

    @pl.when(s == 0)
    def _():
        pltpu.sync_copy(vbuf.at[pl.ds(0, ACCR - NTOT)],
                        acc.at[pl.ds(NTOT, ACCR - NTOT)])

    plsc.subcore_barrier()

    def ebody(g, carry):
        base = c * EP + s * EPT + g * (KD * CHUNK)
        ids = [pltpu.async_copy(edges_hbm.at[pl.ds(base + b * CHUNK, CHUNK)],
                                idx.at[b], semi) for b in range(KD)]
        for d in ids:
            d.wait()
        sds = [pltpu.async_copy(ones_v.at[0], acc.at[idx.at[b]], sema,
                                add=True) for b in range(KD)]
        for d in sds:
            d.wait()
        return carry

    lax.fori_loop(0, GD, ebody, None)
    plsc.subcore_barrier()

    def obody(j, carry):
        cid = j * NSC + s

        @pl.when(cid < NZCH)
        def _():
            pltpu.sync_copy(acc.at[pl.ds(cid * ZCH, ZCH)], vbuf)
            pltpu.sync_copy(vbuf, out_hbm.at[pl.ds(c * NTOT + cid * ZCH, ZCH)])

        return carry

    lax.fori_loop(0, -(-NZCH // NSC), obody, None)


@functools.partial(
    pl.kernel,
    out_type=jax.ShapeDtypeStruct((NTOT, 2, HALF), jnp.float32),
    mesh=_mesh,
    compiler_params=pltpu.CompilerParams(use_tc_tiling_on_sc=False),
    scratch_types=[
        pltpu.VMEM((2 * GSZ,), jnp.int32),
        pltpu.VMEM((2 * KP, CHUNK), jnp.int32),
        pltpu.VMEM((GSZ, HALF), jnp.float32),
        pltpu.VMEM_SHARED((ACCR, HALF), jnp.float32),
        pltpu.SemaphoreType.DMA,
        pltpu.SemaphoreType.DMA,
        pltpu.SemaphoreType.DMA,
    ],
)
def _sc_edge_agg(hn_hbm, srcoff_hbm, dst_hbm, zeros_hbm, out_hbm,
                 sidx, didx, rows, acc, semi, semg, sema):
    c = lax.axis_index("c")
    s = lax.axis_index("s")
    pltpu.sync_copy(zeros_hbm, rows.at[pl.ds(0, CHUNK)])

    def zbody(j, carry):
        cid = j * NSC + s

        @pl.when(cid < ROWCH)
        def _():
            pltpu.sync_copy(rows.at[pl.ds(0, CHUNK)],
                            acc.at[pl.ds(cid * CHUNK, CHUNK)])

        return carry

    lax.fori_loop(0, ROW_ITERS, zbody, None)

    @pl.when(s == 0)
    def _():
        pltpu.sync_copy(rows.at[pl.ds(0, ZTAIL)],
                        acc.at[pl.ds(ROWCH * CHUNK, ZTAIL)])

    plsc.subcore_barrier()
    cbase = c * EP + s * EPT

    def _fire_idx(g, q):
        pltpu.async_copy(srcoff_hbm.at[pl.ds(cbase + g * GSZ, GSZ)],
                         sidx.at[pl.ds(q * GSZ, GSZ)], semi)
        for b in range(KP):
            pltpu.async_copy(
                dst_hbm.at[pl.ds(cbase + g * GSZ + b * CHUNK, CHUNK)],
                didx.at[q * KP + b], semi)

    def _drain_idx(q):
        pltpu.make_async_copy(srcoff_hbm.at[pl.ds(cbase, GSZ)],
                              sidx.at[pl.ds(q * GSZ, GSZ)], semi).wait()
        for b in range(KP):
            pltpu.make_async_copy(dst_hbm.at[pl.ds(cbase, CHUNK)],
                                  didx.at[q * KP + b], semi).wait()

    _fire_idx(0, 0)

    def ebody(j, carry):
        for r in range(2):
            g = 2 * j + r
            q, qn = r, 1 - r
            _drain_idx(q)
            _fire_idx(g + 1, qn)
            gds = [pltpu.async_copy(
                hn_hbm.at[sidx.at[pl.ds(q * GSZ + b * CHUNK, CHUNK)]],
                rows.at[pl.ds(b * CHUNK, CHUNK)], semg) for b in range(KP)]
            for d in gds:
                d.wait()
            sds = [pltpu.async_copy(rows.at[pl.ds(b * CHUNK, CHUNK)],
                                    acc.at[didx.at[q * KP + b]], sema,
                                    add=True) for b in range(KP)]
            for d in sds:
                d.wait()
        return carry

    lax.fori_loop(0, G // 2, ebody, None)
    _drain_idx(0)
    plsc.subcore_barrier()

    def obody(j, carry):
        cid = j * NSC + s

        @pl.when(cid < ROWCH)
        def _():
            r0 = cid * CHUNK
            pltpu.sync_copy(acc.at[pl.ds(r0, CHUNK)], rows.at[pl.ds(0, CHUNK)])
            pltpu.sync_copy(rows.at[pl.ds(0, CHUNK)],
                            out_hbm.at[pl.ds(r0, CHUNK), c])

        return carry

    lax.fori_loop(0, ROW_ITERS, obody, None)

    @pl.when(s == 0)
    def _():
        pltpu.sync_copy(acc.at[pl.ds(ROWCH * CHUNK, ROW_TAIL)],
                        rows.at[pl.ds(0, ROW_TAIL)])
        pltpu.sync_copy(rows.at[pl.ds(0, ROW_TAIL)],
                        out_hbm.at[pl.ds(ROWCH * CHUNK, ROW_TAIL), c])


def _x_body(x_ref, w_ref, b_ref, o_ref):
    o_ref[...] = (jnp.dot(x_ref[...], w_ref[...],
                          preferred_element_type=jnp.float32) + b_ref[...])


def _tc_x(x2, wbd, b2):
    return pl.pallas_call(
        _x_body,
        grid=(XF // XBLK,),
        in_specs=[pl.BlockSpec((XBLK, 2 * IN_DIM), lambda i: (i, 0)),
                  pl.BlockSpec((2 * IN_DIM, HID * 2), lambda i: (0, 0)),
                  pl.BlockSpec((1, HID * 2), lambda i: (0, 0))],
        out_specs=pl.BlockSpec((XBLK, HID * 2), lambda i: (i, 0)),
        out_shape=jax.ShapeDtypeStruct((XF, HID * 2), jnp.float32),
    )(x2, wbd, b2)


def _hn_body(fs_ref, wt_ref, ns_ref, o_ref):
    h = jnp.dot(fs_ref[...], wt_ref[...], preferred_element_type=jnp.float32)
    o_ref[...] = h * ns_ref[...]


def _tc_hn(fs2, wbd, ns2):
    return pl.pallas_call(
        _hn_body,
        grid=(NBLK,),
        in_specs=[pl.BlockSpec((BLK, 2 * HID), lambda i: (i, 0)),
                  pl.BlockSpec((2 * HID, 2 * HID), lambda i: (0, 0)),
                  pl.BlockSpec((BLK, 2 * HID), lambda i: (i, 0))],
        out_specs=pl.BlockSpec((BLK, 2 * HID), lambda i: (i, 0)),
        out_shape=jax.ShapeDtypeStruct((NF, 2 * HID), jnp.float32),
    )(fs2, wbd, ns2)


def _mid_body(agg_ref, nd_ref, b0_ref, wt1_ref, ns_ref, o_ref):
    y = agg_ref[...] * nd_ref[...] + b0_ref[...]
    y = jnp.maximum(y, 0.0)
    o_ref[...] = (jnp.dot(y, wt1_ref[...], preferred_element_type=jnp.float32)
                  * ns_ref[...])


def _tc_mid(agg2, nd2, b02, wbd1, ns2):
    return pl.pallas_call(
        _mid_body,
        grid=(NBLK,),
        in_specs=[pl.BlockSpec((BLK, 2 * HID), lambda i: (i, 0)),
                  pl.BlockSpec((BLK, 2 * HID), lambda i: (i, 0)),
                  pl.BlockSpec((1, 2 * HID), lambda i: (0, 0)),
                  pl.BlockSpec((2 * HID, 2 * HID), lambda i: (0, 0)),
                  pl.BlockSpec((BLK, 2 * HID), lambda i: (i, 0))],
        out_specs=pl.BlockSpec((BLK, 2 * HID), lambda i: (i, 0)),
        out_shape=jax.ShapeDtypeStruct((NF, 2 * HID), jnp.float32),
    )(agg2, nd2, b02, wbd1, ns2)


def _cr_body(agg_ref, nd_ref, b1_ref, fs_ref, cr_ref, ps_ref):
    cr = agg_ref[...] * nd_ref[...] + b1_ref[...] + fs_ref[...]
    cr_ref[...] = cr
    ps_ref[...] = jnp.stack([jnp.sum(cr), jnp.sum(cr * cr)]).reshape(1, 1, 2)


def _tc_cr(agg2, nd2, b12, fs2):
    return pl.pallas_call(
        _cr_body,
        grid=(NBLK,),
        in_specs=[pl.BlockSpec((BLK, 2 * HID), lambda i: (i, 0)),
                  pl.BlockSpec((BLK, 2 * HID), lambda i: (i, 0)),
                  pl.BlockSpec((1, 2 * HID), lambda i: (0, 0)),
                  pl.BlockSpec((BLK, 2 * HID), lambda i: (i, 0))],
        out_specs=[pl.BlockSpec((BLK, 2 * HID), lambda i: (i, 0)),
                   pl.BlockSpec((1, 1, 2), lambda i: (i, 0, 0))],
        out_shape=[jax.ShapeDtypeStruct((NF, 2 * HID), jnp.float32),
                   jax.ShapeDtypeStruct((NBLK, 1, 2), jnp.float32)],
    )(agg2, nd2, b12, fs2)


def _fin_body(crl_ref, ps_ref, o_ref):
    tot = jnp.sum(ps_ref[...], axis=(0, 1))
    mu = tot[0] / TOTEL
    var = tot[1] / TOTEL - mu * mu
    rs = lax.rsqrt(var + 1e-5)
    o_ref[...] = (crl_ref[...] - mu) * rs


def _tc_fin(crl, ps):
    return pl.pallas_call(
        _fin_body,
        out_shape=jax.ShapeDtypeStruct((B * NN, HID), jnp.float32),
    )(crl, ps)


def _head_body(v_ref, w1_ref, b1_ref, w2_ref, b2_ref, o_ref):
    v = v_ref[...]
    w1 = w1_ref[...]
    b1 = b1_ref[...]
    w2 = w2_ref[...]
    b2 = b2_ref[...]
    cols = []
    for p in range(PRED):
        m = jnp.maximum(v * w1[0, p] + b1[0, p], 0.0)
        cols.append(jnp.dot(m, w2, preferred_element_type=jnp.float32))
    o_ref[...] = jnp.concatenate(cols, axis=1) + b2[0, 0]


HBLK = 1000


def _tc_head(v, w1row, b1row, w2col, b2):
    return pl.pallas_call(
        _head_body,
        grid=(B * NN // HBLK,),
        in_specs=[pl.BlockSpec((HBLK, HID), lambda i: (i, 0)),
                  pl.BlockSpec((1, PRED), lambda i: (0, 0)),
                  pl.BlockSpec((1, PRED), lambda i: (0, 0)),
                  pl.BlockSpec((HID, 1), lambda i: (0, 0)),
                  pl.BlockSpec((1, 1), lambda i: (0, 0))],
        out_specs=pl.BlockSpec((HBLK, PRED), lambda i: (i, 0)),
        out_shape=jax.ShapeDtypeStruct((B * NN, PRED), jnp.float32),
    )(v, w1row, b1row, w2col, b2)


def _norm_body(deg_ref, o_ref):
    o_ref[...] = lax.rsqrt(jnp.maximum(deg_ref[...], 1.0))


def _tc_norms(deg2):
    return pl.pallas_call(
        _norm_body,
        out_shape=jax.ShapeDtypeStruct((2, NTOT), jnp.float32),
    )(deg2)


def _blockdiag(w):
    n, m = w.shape
    z = jnp.zeros((n, m), jnp.float32)
    return jnp.concatenate([jnp.concatenate([w, z], axis=1),
                            jnp.concatenate([z, w], axis=1)], axis=0)


def kernel(inputs, edge_index, W_in, b_in, Wg0, bg0, Wg1, bg1,
           Wo1, bo1, Wo2, bo2):
    src = edge_index[0]
    dst = edge_index[1]

    x2 = _tc_x(inputs.reshape(XF, 2 * IN_DIM), _blockdiag(W_in.T),
               jnp.tile(b_in, 2).reshape(1, 2 * HID))
    x4 = x2.reshape(B, HIS, NN, HID)

    z1 = jnp.zeros((ZCH,), jnp.float32)
    z2 = jnp.zeros((CHUNK, HALF), jnp.float32)
    pad0 = jnp.zeros((PAD,), jnp.int32)
    padN = jnp.full((PAD,), NTOT, jnp.int32)
    ext0 = jnp.zeros((EXT,), jnp.int32)
    extN = jnp.full((EXT,), NTOT, jnp.int32)
    srcoff = jnp.concatenate([2 * src, pad0, 2 * src + 1, pad0, ext0])
    dst2 = jnp.concatenate([dst, padN, dst, padN, extN])
    deg = _sc_degrees(jnp.concatenate([src, padN, dst, padN]), z1)
    norms = _tc_norms(deg.reshape(2, NTOT))
    ns2 = jnp.repeat(norms[0], HID).reshape(NF, 2 * HID)
    nd2 = jnp.repeat(norms[1], HID).reshape(NF, 2 * HID)

    w0bd = _blockdiag(Wg0.T)
    w1bd = _blockdiag(Wg1.T)
    b02 = jnp.tile(bg0, 2).reshape(1, 2 * HID)
    b12 = jnp.tile(bg1, 2).reshape(1, 2 * HID)

    chp = [S, 2 * S - 1, HIS]
    left = 0
    lastn = None
    for r, right in enumerate(chp):
        if r == 0:
            fs2 = x4[:, 0:S].reshape(NF, 2 * HID)
        else:
            fs2 = jnp.concatenate(
                [lastn.reshape(B, 1, NN, HID), x4[:, left:right]],
                axis=1).reshape(NF, 2 * HID)
        hn1 = _tc_hn(fs2, w0bd, ns2)
        agg1 = _sc_edge_agg(hn1.reshape(2 * NTOT, HALF), srcoff, dst2, z2)
        hn2 = _tc_mid(agg1.reshape(NF, 2 * HID), nd2, b02, w1bd, ns2)
        agg2 = _sc_edge_agg(hn2.reshape(2 * NTOT, HALF), srcoff, dst2, z2)
        cr, ps = _tc_cr(agg2.reshape(NF, 2 * HID), nd2, b12, fs2)
        crl = cr.reshape(B, S, NN, HID)[:, S - 1].reshape(B * NN, HID)
        lastn = _tc_fin(crl, ps)
        left = right

    o = _tc_head(lastn, Wo1.reshape(1, PRED), bo1.reshape(1, PRED),
                 Wo2.reshape(HID, 1), bo2.reshape(1, 1))
    return o.reshape(B, NN, PRED).transpose(0, 2, 1)[..., None]
